# baseline (speedup 1.0000x reference)
import functools

import jax
import jax.numpy as jnp
from jax import lax
from jax.experimental import pallas as pl
from jax.experimental.pallas import tpu as pltpu
from jax.experimental.pallas import tpu_sc as plsc

B, S = 2, 2048
T = B * S
D_MODEL = 1024
N_FEAT = 32
PROTO_DIM = 256
D_FEMB = 128
D_RH = 256
E = 8
K = 2
DH = 1024
D_XIN = D_MODEL + D_FEMB
D_XPAD = 1280
EPAD = 128
NEG = -1e30

BLK_T = 256
BLK = 256
NP = T * K
P_MAX = NP + E * BLK
NB = P_MAX // BLK

NW = 32
GCH = 80
CT = 16


def _preamble_body(hid, ft, proto, w_hctx, w_fctx, w_feat, b_feat,
                   w_r1h, w_r1f, b_r1, w_r2p, b_r2p,
                   glp, gwp, xcat, meta):
    proto_row = proto[0]
    hc = hid[...] + jnp.dot(proto_row, w_hctx[...],
                            preferred_element_type=jnp.float32)
    fc = ft[...] + jnp.dot(proto_row, w_fctx[...],
                           preferred_element_type=jnp.float32)
    fe = jax.nn.relu(jnp.dot(fc, w_feat[...],
                             preferred_element_type=jnp.float32) + b_feat[...])
    rh = jax.nn.relu(
        jnp.dot(hc, w_r1h[...], preferred_element_type=jnp.float32)
        + jnp.dot(fc, w_r1f[...], preferred_element_type=jnp.float32)
        + b_r1[...])
    lg = jnp.dot(rh, w_r2p[...], preferred_element_type=jnp.float32) + b_r2p[...]
    glp[...] = lg

    lanes = jax.lax.broadcasted_iota(jnp.int32, (BLK_T, EPAD), 1)
    v1 = jnp.max(lg, axis=1, keepdims=True)
    i1 = jnp.min(jnp.where(lg == v1, lanes, EPAD), axis=1, keepdims=True)
    lg2 = jnp.where(lanes == i1, NEG, lg)
    v2 = jnp.max(lg2, axis=1, keepdims=True)
    i2 = jnp.min(jnp.where(lg2 == v2, lanes, EPAD), axis=1, keepdims=True)
    w1 = 1.0 / (1.0 + jnp.exp(v2 - v1))
    w2 = 1.0 - w1
    gw = jnp.where(lanes == i1, w1, 0.0) + jnp.where(lanes == i2, w2, 0.0)
    gwp[...] = gw

    xcat[:, :D_MODEL] = hc.astype(jnp.bfloat16)
    xcat[:, D_MODEL:D_XIN] = fe.astype(jnp.bfloat16)
    xcat[:, D_XIN:] = jnp.zeros((BLK_T, D_XPAD - D_XIN), jnp.bfloat16)
    meta[...] = (jnp.where(lanes == 0, i1.astype(jnp.float32), 0.0)
                 + jnp.where(lanes == 1, i2.astype(jnp.float32), 0.0)
                 + jnp.where(lanes == 2, w1, 0.0)
                 + jnp.where(lanes == 3, w2, 0.0))


def _preamble(hid, ft, proto, W_hctx, W_fctx, W_feat, b_feat,
              w_r1h, w_r1f, b_r1, w_r2p, b_r2p):
    nblk = T // BLK_T
    const = lambda *shp: pl.BlockSpec(shp, lambda i: (0,) * len(shp))
    grid_spec = pl.GridSpec(
        grid=(nblk,),
        in_specs=[
            pl.BlockSpec((BLK_T, D_MODEL), lambda i: (i, 0)),
            pl.BlockSpec((BLK_T, N_FEAT), lambda i: (i, 0)),
            pl.BlockSpec((1, 1, PROTO_DIM), lambda i: (i // (S // BLK_T), 0, 0)),
            const(PROTO_DIM, D_MODEL),
            const(PROTO_DIM, N_FEAT),
            const(N_FEAT, D_FEMB),
            const(1, D_FEMB),
            const(D_MODEL, D_RH),
            const(N_FEAT, D_RH),
            const(1, D_RH),
            const(D_RH, EPAD),
            const(1, EPAD),
        ],
        out_specs=[
            pl.BlockSpec((BLK_T, EPAD), lambda i: (i, 0)),
            pl.BlockSpec((BLK_T, EPAD), lambda i: (i, 0)),
            pl.BlockSpec((BLK_T, D_XPAD), lambda i: (i, 0)),
            pl.BlockSpec((BLK_T, EPAD), lambda i: (i, 0)),
        ],
    )
    return pl.pallas_call(
        _preamble_body,
        grid_spec=grid_spec,
        out_shape=[
            jax.ShapeDtypeStruct((T, EPAD), jnp.float32),
            jax.ShapeDtypeStruct((T, EPAD), jnp.float32),
            jax.ShapeDtypeStruct((T, D_XPAD), jnp.bfloat16),
            jax.ShapeDtypeStruct((T, EPAD), jnp.float32),
        ],
    )(hid, ft, proto, W_hctx, W_fctx, W_feat, b_feat,
      w_r1h, w_r1f, b_r1, w_r2p, b_r2p)


def _sc_mesh():
    return plsc.VectorSubcoreMesh(core_axis_name="c", subcore_axis_name="s")


def _sc_gather(row_tok, xcat):
    pk = D_XPAD // 2
    rw = P_MAX // NW
    nch = rw // GCH

    @functools.partial(
        pl.kernel,
        mesh=_sc_mesh(),
        out_type=jax.ShapeDtypeStruct((P_MAX, pk), jnp.float32),
        scratch_types=[
            pltpu.VMEM((rw,), jnp.int32),
            pltpu.VMEM((GCH, pk), jnp.float32),
            pltpu.VMEM((GCH, pk), jnp.float32),
            pltpu.SemaphoreType.DMA,
            pltpu.SemaphoreType.DMA,
            pltpu.SemaphoreType.DMA,
            pltpu.SemaphoreType.DMA,
        ],
    )
    def gather_k(tok_hbm, xcat_hbm, out_hbm, idxs, buf0, buf1,
                 g0, g1, s0, s1):
        wid = lax.axis_index("s") * 2 + lax.axis_index("c")
        base = wid * rw
        bufs, gsems, ssems = [buf0, buf1], [g0, g1], [s0, s1]
        pltpu.sync_copy(tok_hbm.at[pl.ds(base, rw)], idxs)

        def issue(c):
            return pltpu.async_copy(
                xcat_hbm.at[idxs.at[pl.ds(c * GCH, GCH)]],
                bufs[c % 2], gsems[c % 2])

        g = [None] * nch
        s = [None] * nch
        g[0] = issue(0)
        for c in range(nch):
            if c + 1 < nch:
                g[c + 1] = issue(c + 1)
            g[c].wait()
            if c >= 2:
                s[c - 2].wait()
            s[c] = pltpu.async_copy(
                bufs[c % 2], out_hbm.at[pl.ds(base + c * GCH, GCH)],
                ssems[c % 2])
        for c in range(max(nch - 2, 0), nch):
            s[c].wait()

    return gather_k(row_tok, xcat)


def _gmm_body(be_ref, xs, rw, w1, b1, w2, b2, ys):
    del be_ref
    xb = xs[...]
    h1 = jax.nn.relu(jnp.dot(xb, w1[0], preferred_element_type=jnp.float32)
                     + b1[0])
    y = jnp.dot(h1.astype(jnp.bfloat16), w2[0],
                preferred_element_type=jnp.float32) + b2[0]
    ys[...] = y * rw[...]


def _grouped_matmul(block_expert, xs, row_w, w_e1, b_e1, w_e2, b_e2):
    grid_spec = pltpu.PrefetchScalarGridSpec(
        num_scalar_prefetch=1,
        grid=(NB,),
        in_specs=[
            pl.BlockSpec((BLK, D_XPAD), lambda j, be: (j, 0)),
            pl.BlockSpec((BLK, 1), lambda j, be: (j, 0)),
            pl.BlockSpec((1, D_XPAD, DH), lambda j, be: (be[j], 0, 0)),
            pl.BlockSpec((1, 1, DH), lambda j, be: (be[j], 0, 0)),
            pl.BlockSpec((1, DH, D_MODEL), lambda j, be: (be[j], 0, 0)),
            pl.BlockSpec((1, 1, D_MODEL), lambda j, be: (be[j], 0, 0)),
        ],
        out_specs=pl.BlockSpec((BLK, D_MODEL), lambda j, be: (j, 0)),
    )
    return pl.pallas_call(
        _gmm_body,
        grid_spec=grid_spec,
        out_shape=jax.ShapeDtypeStruct((P_MAX, D_MODEL), jnp.float32),
    )(block_expert, xs, row_w, w_e1, b_e1, w_e2, b_e2)


def _sc_combine(dest, ys):
    tw = T // NW
    nch = tw // CT

    @functools.partial(
        pl.kernel,
        mesh=_sc_mesh(),
        out_type=jax.ShapeDtypeStruct((T, D_MODEL), jnp.float32),
        scratch_types=[
            pltpu.VMEM((K * tw,), jnp.int32),
            pltpu.VMEM((K * CT, D_MODEL), jnp.float32),
            pltpu.VMEM((K * CT, D_MODEL), jnp.float32),
            pltpu.VMEM((CT, D_MODEL), jnp.float32),
            pltpu.VMEM((CT, D_MODEL), jnp.float32),
            pltpu.SemaphoreType.DMA,
            pltpu.SemaphoreType.DMA,
            pltpu.SemaphoreType.DMA,
            pltpu.SemaphoreType.DMA,
        ],
    )
    def combine_k(dest_hbm, ys_hbm, out_hbm, idxs, buf0, buf1,
                  ob0, ob1, g0, g1, s0, s1):
        wid = lax.axis_index("s") * 2 + lax.axis_index("c")
        base = wid * tw
        bufs, obs, gsems, ssems = [buf0, buf1], [ob0, ob1], [g0, g1], [s0, s1]
        pltpu.sync_copy(dest_hbm.at[pl.ds(K * base, K * tw)], idxs)

        def issue(c):
            return pltpu.async_copy(
                ys_hbm.at[idxs.at[pl.ds(c * K * CT, K * CT)]],
                bufs[c % 2], gsems[c % 2])

        g = [None] * nch
        s = [None] * nch
        g[0] = issue(0)
        for c in range(nch):
            if c + 1 < nch:
                g[c + 1] = issue(c + 1)
            g[c].wait()
            if c >= 2:
                s[c - 2].wait()
            buf, ob = bufs[c % 2], obs[c % 2]

            def row_add(i, _, buf=buf, ob=ob):
                for d in range(D_MODEL // 16):
                    sl = slice(d * 16, d * 16 + 16)
                    ob[i, sl] = buf[2 * i, sl] + buf[2 * i + 1, sl]
                return 0

            lax.fori_loop(0, CT, row_add, 0)
            s[c] = pltpu.async_copy(
                ob, out_hbm.at[pl.ds(base + c * CT, CT)], ssems[c % 2])
        for c in range(max(nch - 2, 0), nch):
            s[c].wait()

    return combine_k(dest, ys)


def kernel(hidden, feat, proto_context, W_hctx, W_fctx, W_feat, b_feat,
           W_r1, b_r1, W_r2, b_r2, W_e1, b_e1, W_e2, b_e2):
    hid = hidden.reshape(T, D_MODEL)
    ft = feat.reshape(T, N_FEAT)
    proto = proto_context.reshape(B, 1, PROTO_DIM)
    w_r1h = W_r1[:D_MODEL]
    w_r1f = W_r1[D_MODEL:]
    w_r2p = jnp.zeros((D_RH, EPAD), jnp.float32).at[:, :E].set(W_r2)
    b_r2p = jnp.full((1, EPAD), NEG, jnp.float32).at[0, :E].set(b_r2)

    glp, gwp, xcat, meta = _preamble(
        hid, ft, proto, W_hctx, W_fctx, W_feat, b_feat.reshape(1, D_FEMB),
        w_r1h, w_r1f, b_r1.reshape(1, D_RH), w_r2p, b_r2p)

    e1 = meta[:, 0].astype(jnp.int32)
    e2 = meta[:, 1].astype(jnp.int32)
    ep = jnp.stack([e1, e2], axis=1).reshape(-1)
    wp = jnp.stack([meta[:, 2], meta[:, 3]], axis=1).reshape(-1)
    oh = (ep[:, None] == jnp.arange(E, dtype=jnp.int32)[None, :]).astype(jnp.int32)
    csum = jnp.cumsum(oh, axis=0)
    rank = jnp.take_along_axis(csum, ep[:, None], axis=1)[:, 0] - 1
    counts = csum[-1]
    padded = ((counts + BLK - 1) // BLK) * BLK
    ends = jnp.cumsum(padded)
    starts = ends - padded
    dest = (starts[ep] + rank).astype(jnp.int32)
    tok = jnp.arange(NP, dtype=jnp.int32) // K
    row_tok = jnp.zeros((P_MAX,), jnp.int32).at[dest].set(tok)
    row_w = jnp.zeros((P_MAX, 1), jnp.float32).at[dest, 0].set(wp)
    block_expert = jnp.clip(
        jnp.searchsorted(ends, jnp.arange(NB, dtype=jnp.int32) * BLK,
                         side="right"), 0, E - 1).astype(jnp.int32)

    xcat_pk = lax.bitcast_convert_type(
        xcat.reshape(T, D_XPAD // 2, 2), jnp.float32)
    xs_pk = _sc_gather(row_tok, xcat_pk)
    xs = lax.bitcast_convert_type(xs_pk, jnp.bfloat16).reshape(P_MAX, D_XPAD)
    w_e1p = jnp.zeros((E, D_XPAD, DH), jnp.bfloat16).at[:, :D_XIN, :].set(
        W_e1.astype(jnp.bfloat16))
    ys = _grouped_matmul(block_expert, xs, row_w,
                         w_e1p, b_e1.reshape(E, 1, DH),
                         W_e2.astype(jnp.bfloat16), b_e2.reshape(E, 1, D_MODEL))
    sd = _sc_combine(dest, ys)

    stage_delta = sd.reshape(B, S, D_MODEL)
    gate_logits = glp[:, :E].reshape(B, S, E)
    gate_weights = gwp[:, :E].reshape(B, S, E)
    return stage_delta, gate_weights, gate_logits


# speedup vs baseline: 1.7483x; 1.7483x over previous
import functools

import jax
import jax.numpy as jnp
from jax import lax
from jax.experimental import pallas as pl
from jax.experimental.pallas import tpu as pltpu
from jax.experimental.pallas import tpu_sc as plsc

B, S = 2, 2048
T = B * S
D_MODEL = 1024
N_FEAT = 32
PROTO_DIM = 256
D_FEMB = 128
D_RH = 256
E = 8
K = 2
DH = 1024
D_XIN = D_MODEL + D_FEMB
D_XPAD = 1280
EPAD = 128
NEG = -1e30

BLK_T = 256
BLK = 256
NP = T * K
P_MAX = NP + E * BLK
NB = P_MAX // BLK

NW = 32
GCH = 80
CT = 16


def _preamble_body(hid, ft, proto, w_hctx, w_fctx, w_feat, b_feat,
                   w_r1h, w_r1f, b_r1, w_r2p, b_r2p,
                   glp, gwp, xcat, meta):
    proto_row = proto[0]
    hc = hid[...] + jnp.dot(proto_row, w_hctx[...],
                            preferred_element_type=jnp.float32)
    fc = ft[...] + jnp.dot(proto_row, w_fctx[...],
                           preferred_element_type=jnp.float32)
    fe = jax.nn.relu(jnp.dot(fc, w_feat[...],
                             preferred_element_type=jnp.float32) + b_feat[...])
    rh = jax.nn.relu(
        jnp.dot(hc, w_r1h[...], preferred_element_type=jnp.float32)
        + jnp.dot(fc, w_r1f[...], preferred_element_type=jnp.float32)
        + b_r1[...])
    lg = jnp.dot(rh, w_r2p[...], preferred_element_type=jnp.float32) + b_r2p[...]
    glp[...] = lg

    lanes = jax.lax.broadcasted_iota(jnp.int32, (BLK_T, EPAD), 1)
    v1 = jnp.max(lg, axis=1, keepdims=True)
    i1 = jnp.min(jnp.where(lg == v1, lanes, EPAD), axis=1, keepdims=True)
    lg2 = jnp.where(lanes == i1, NEG, lg)
    v2 = jnp.max(lg2, axis=1, keepdims=True)
    i2 = jnp.min(jnp.where(lg2 == v2, lanes, EPAD), axis=1, keepdims=True)
    w1 = 1.0 / (1.0 + jnp.exp(v2 - v1))
    w2 = 1.0 - w1
    gw = jnp.where(lanes == i1, w1, 0.0) + jnp.where(lanes == i2, w2, 0.0)
    gwp[...] = gw

    def bf16bits(x):
        b = jax.lax.bitcast_convert_type(x, jnp.int32)
        r = (b + 0x7FFF + ((b >> 16) & 1)) >> 16
        return r & 0xFFFF

    xcat[:, 0:384] = (bf16bits(hc[:, 0:384])
                      | (bf16bits(hc[:, 640:1024]) << 16))
    xcat[:, 384:512] = bf16bits(hc[:, 384:512]) | (bf16bits(fe) << 16)
    xcat[:, 512:640] = bf16bits(hc[:, 512:640])
    meta[...] = (jnp.where(lanes == 0, i1.astype(jnp.float32), 0.0)
                 + jnp.where(lanes == 1, i2.astype(jnp.float32), 0.0)
                 + jnp.where(lanes == 2, w1, 0.0)
                 + jnp.where(lanes == 3, w2, 0.0))


def _preamble(hid, ft, proto, W_hctx, W_fctx, W_feat, b_feat,
              w_r1h, w_r1f, b_r1, w_r2p, b_r2p):
    nblk = T // BLK_T
    const = lambda *shp: pl.BlockSpec(shp, lambda i: (0,) * len(shp))
    grid_spec = pl.GridSpec(
        grid=(nblk,),
        in_specs=[
            pl.BlockSpec((BLK_T, D_MODEL), lambda i: (i, 0)),
            pl.BlockSpec((BLK_T, N_FEAT), lambda i: (i, 0)),
            pl.BlockSpec((1, 1, PROTO_DIM), lambda i: (i // (S // BLK_T), 0, 0)),
            const(PROTO_DIM, D_MODEL),
            const(PROTO_DIM, N_FEAT),
            const(N_FEAT, D_FEMB),
            const(1, D_FEMB),
            const(D_MODEL, D_RH),
            const(N_FEAT, D_RH),
            const(1, D_RH),
            const(D_RH, EPAD),
            const(1, EPAD),
        ],
        out_specs=[
            pl.BlockSpec((BLK_T, EPAD), lambda i: (i, 0)),
            pl.BlockSpec((BLK_T, EPAD), lambda i: (i, 0)),
            pl.BlockSpec((BLK_T, D_XPAD // 2), lambda i: (i, 0)),
            pl.BlockSpec((BLK_T, EPAD), lambda i: (i, 0)),
        ],
    )
    return pl.pallas_call(
        _preamble_body,
        grid_spec=grid_spec,
        out_shape=[
            jax.ShapeDtypeStruct((T, EPAD), jnp.float32),
            jax.ShapeDtypeStruct((T, EPAD), jnp.float32),
            jax.ShapeDtypeStruct((T, D_XPAD // 2), jnp.int32),
            jax.ShapeDtypeStruct((T, EPAD), jnp.float32),
        ],
    )(hid, ft, proto, W_hctx, W_fctx, W_feat, b_feat,
      w_r1h, w_r1f, b_r1, w_r2p, b_r2p)


def _sc_mesh():
    return plsc.VectorSubcoreMesh(core_axis_name="c", subcore_axis_name="s")


def _sc_gather(row_tok, xcat):
    pk = D_XPAD // 2
    rw = P_MAX // NW
    nch = rw // GCH

    @functools.partial(
        pl.kernel,
        mesh=_sc_mesh(),
        out_type=jax.ShapeDtypeStruct((P_MAX, pk), jnp.int32),
        scratch_types=[
            pltpu.VMEM((rw,), jnp.int32),
            pltpu.VMEM((GCH, pk), jnp.int32),
            pltpu.VMEM((GCH, pk), jnp.int32),
            pltpu.SemaphoreType.DMA,
            pltpu.SemaphoreType.DMA,
            pltpu.SemaphoreType.DMA,
            pltpu.SemaphoreType.DMA,
        ],
    )
    def gather_k(tok_hbm, xcat_hbm, out_hbm, idxs, buf0, buf1,
                 g0, g1, s0, s1):
        wid = lax.axis_index("s") * 2 + lax.axis_index("c")
        base = wid * rw
        bufs, gsems, ssems = [buf0, buf1], [g0, g1], [s0, s1]
        pltpu.sync_copy(tok_hbm.at[pl.ds(base, rw)], idxs)

        def issue(c):
            return pltpu.async_copy(
                xcat_hbm.at[idxs.at[pl.ds(c * GCH, GCH)]],
                bufs[c % 2], gsems[c % 2])

        g = [None] * nch
        s = [None] * nch
        g[0] = issue(0)
        for c in range(nch):
            if c + 1 < nch:
                g[c + 1] = issue(c + 1)
            g[c].wait()
            if c >= 2:
                s[c - 2].wait()
            s[c] = pltpu.async_copy(
                bufs[c % 2], out_hbm.at[pl.ds(base + c * GCH, GCH)],
                ssems[c % 2])
        for c in range(max(nch - 2, 0), nch):
            s[c].wait()

    return gather_k(row_tok, xcat)


def _gmm_body(be_ref, xs, rw, w1, b1, w2, b2, ys):
    del be_ref
    xi = xs[...]
    xlo = jax.lax.bitcast_convert_type(xi << 16, jnp.float32).astype(jnp.bfloat16)
    xhi = jax.lax.bitcast_convert_type(
        xi & jnp.int32(-65536), jnp.float32).astype(jnp.bfloat16)
    half = D_XPAD // 2
    h1 = jax.nn.relu(
        jnp.dot(xlo, w1[0, :half, :], preferred_element_type=jnp.float32)
        + jnp.dot(xhi, w1[0, half:, :], preferred_element_type=jnp.float32)
        + b1[0])
    y = jnp.dot(h1.astype(jnp.bfloat16), w2[0],
                preferred_element_type=jnp.float32) + b2[0]
    ys[...] = y * rw[...]


def _grouped_matmul(block_expert, xs, row_w, w_e1, b_e1, w_e2, b_e2):
    grid_spec = pltpu.PrefetchScalarGridSpec(
        num_scalar_prefetch=1,
        grid=(NB,),
        in_specs=[
            pl.BlockSpec((BLK, D_XPAD // 2), lambda j, be: (j, 0)),
            pl.BlockSpec((BLK, 1), lambda j, be: (j, 0)),
            pl.BlockSpec((1, D_XPAD, DH), lambda j, be: (be[j], 0, 0)),
            pl.BlockSpec((1, 1, DH), lambda j, be: (be[j], 0, 0)),
            pl.BlockSpec((1, DH, D_MODEL), lambda j, be: (be[j], 0, 0)),
            pl.BlockSpec((1, 1, D_MODEL), lambda j, be: (be[j], 0, 0)),
        ],
        out_specs=pl.BlockSpec((BLK, D_MODEL), lambda j, be: (j, 0)),
    )
    return pl.pallas_call(
        _gmm_body,
        grid_spec=grid_spec,
        out_shape=jax.ShapeDtypeStruct((P_MAX, D_MODEL), jnp.float32),
    )(block_expert, xs, row_w, w_e1, b_e1, w_e2, b_e2)


def _sc_combine(dest, ys):
    tw = T // NW
    nch = tw // CT

    @functools.partial(
        pl.kernel,
        mesh=_sc_mesh(),
        out_type=jax.ShapeDtypeStruct((T, D_MODEL), jnp.float32),
        scratch_types=[
            pltpu.VMEM((K * tw,), jnp.int32),
            pltpu.VMEM((K * CT, D_MODEL), jnp.float32),
            pltpu.VMEM((K * CT, D_MODEL), jnp.float32),
            pltpu.VMEM((CT, D_MODEL), jnp.float32),
            pltpu.VMEM((CT, D_MODEL), jnp.float32),
            pltpu.SemaphoreType.DMA,
            pltpu.SemaphoreType.DMA,
            pltpu.SemaphoreType.DMA,
            pltpu.SemaphoreType.DMA,
        ],
    )
    def combine_k(dest_hbm, ys_hbm, out_hbm, idxs, buf0, buf1,
                  ob0, ob1, g0, g1, s0, s1):
        wid = lax.axis_index("s") * 2 + lax.axis_index("c")
        base = wid * tw
        bufs, obs, gsems, ssems = [buf0, buf1], [ob0, ob1], [g0, g1], [s0, s1]
        pltpu.sync_copy(dest_hbm.at[pl.ds(K * base, K * tw)], idxs)

        def issue(c):
            return pltpu.async_copy(
                ys_hbm.at[idxs.at[pl.ds(c * K * CT, K * CT)]],
                bufs[c % 2], gsems[c % 2])

        g = [None] * nch
        s = [None] * nch
        g[0] = issue(0)
        for c in range(nch):
            if c + 1 < nch:
                g[c + 1] = issue(c + 1)
            g[c].wait()
            if c >= 2:
                s[c - 2].wait()
            buf, ob = bufs[c % 2], obs[c % 2]

            def row_add(i, _, buf=buf, ob=ob):
                for d in range(D_MODEL // 16):
                    sl = slice(d * 16, d * 16 + 16)
                    ob[i, sl] = buf[2 * i, sl] + buf[2 * i + 1, sl]
                return 0

            lax.fori_loop(0, CT, row_add, 0)
            s[c] = pltpu.async_copy(
                ob, out_hbm.at[pl.ds(base + c * CT, CT)], ssems[c % 2])
        for c in range(max(nch - 2, 0), nch):
            s[c].wait()

    return combine_k(dest, ys)


def kernel(hidden, feat, proto_context, W_hctx, W_fctx, W_feat, b_feat,
           W_r1, b_r1, W_r2, b_r2, W_e1, b_e1, W_e2, b_e2):
    hid = hidden.reshape(T, D_MODEL)
    ft = feat.reshape(T, N_FEAT)
    proto = proto_context.reshape(B, 1, PROTO_DIM)
    w_r1h = W_r1[:D_MODEL]
    w_r1f = W_r1[D_MODEL:]
    w_r2p = jnp.zeros((D_RH, EPAD), jnp.float32).at[:, :E].set(W_r2)
    b_r2p = jnp.full((1, EPAD), NEG, jnp.float32).at[0, :E].set(b_r2)

    glp, gwp, xcat, meta = _preamble(
        hid, ft, proto, W_hctx, W_fctx, W_feat, b_feat.reshape(1, D_FEMB),
        w_r1h, w_r1f, b_r1.reshape(1, D_RH), w_r2p, b_r2p)

    e1 = meta[:, 0].astype(jnp.int32)
    e2 = meta[:, 1].astype(jnp.int32)
    ep = jnp.stack([e1, e2], axis=1).reshape(-1)
    wp = jnp.stack([meta[:, 2], meta[:, 3]], axis=1).reshape(-1)
    oh = (ep[:, None] == jnp.arange(E, dtype=jnp.int32)[None, :]).astype(jnp.int32)
    csum = jnp.cumsum(oh, axis=0)
    rank = jnp.take_along_axis(csum, ep[:, None], axis=1)[:, 0] - 1
    counts = csum[-1]
    padded = ((counts + BLK - 1) // BLK) * BLK
    ends = jnp.cumsum(padded)
    starts = ends - padded
    dest = (starts[ep] + rank).astype(jnp.int32)
    tok = jnp.arange(NP, dtype=jnp.int32) // K
    row_tok = jnp.zeros((P_MAX,), jnp.int32).at[dest].set(tok)
    row_w = jnp.zeros((P_MAX, 1), jnp.float32).at[dest, 0].set(wp)
    block_expert = jnp.clip(
        jnp.searchsorted(ends, jnp.arange(NB, dtype=jnp.int32) * BLK,
                         side="right"), 0, E - 1).astype(jnp.int32)

    xs = _sc_gather(row_tok, xcat)
    w_e1p = jnp.zeros((E, D_XPAD, DH), jnp.bfloat16).at[:, :D_XIN, :].set(
        W_e1.astype(jnp.bfloat16))
    ys = _grouped_matmul(block_expert, xs, row_w,
                         w_e1p, b_e1.reshape(E, 1, DH),
                         W_e2.astype(jnp.bfloat16), b_e2.reshape(E, 1, D_MODEL))
    sd = _sc_combine(dest, ys)

    stage_delta = sd.reshape(B, S, D_MODEL)
    gate_logits = glp[:, :E].reshape(B, S, E)
    gate_weights = gwp[:, :E].reshape(B, S, E)
    return stage_delta, gate_weights, gate_logits
